# Initial kernel scaffold; baseline (speedup 1.0000x reference)
import jax
import jax.numpy as jnp
from jax.experimental import pallas as pl


def kernel(target_center_ids, bank, count):
    raise NotImplementedError("write your pallas kernel here")


# speedup vs baseline: 2.3764x; 2.3764x over previous
import functools

import jax
import jax.numpy as jnp
import numpy as np
from jax import lax
from jax.experimental import pallas as pl
from jax.experimental.pallas import tpu as pltpu
from jax.experimental.pallas import tpu_sc as plsc

N_CENTERS = 10000
CAP = 100
FDIM = 64
BATCH_N = 16384


C_BLK = 40


def _means_body(count_ref, bank_ref, out_ref):
    cnt = count_ref[...]
    cntf = cnt.astype(jnp.float32)
    pos = lax.broadcasted_iota(jnp.int32, (C_BLK, CAP), 1)
    w = jnp.where(pos < cnt, 1.0, 0.0) / jnp.maximum(cntf, 1.0)
    out_ref[...] = jnp.sum(bank_ref[...] * w[:, :, None], axis=1)


def _compute_means(bank, count):
    return pl.pallas_call(
        _means_body,
        grid=(N_CENTERS // C_BLK,),
        in_specs=[
            pl.BlockSpec((C_BLK, 1), lambda i: (i, 0)),
            pl.BlockSpec((C_BLK, CAP, FDIM), lambda i: (i, 0, 0)),
        ],
        out_specs=pl.BlockSpec((C_BLK, FDIM), lambda i: (i, 0)),
        out_shape=jax.ShapeDtypeStruct((N_CENTERS, FDIM), jnp.float32),
    )(count.reshape(N_CENTERS, 1), bank)


_NC = 2
_NS = 16
_NW = _NC * _NS
_BPW = BATCH_N // _NW
_CHUNK = 128
_NCH = _BPW // _CHUNK

@functools.cache
def _build_gather_rows():
    mesh = plsc.VectorSubcoreMesh(core_axis_name="c", subcore_axis_name="s")
    return functools.partial(
        pl.kernel,
        mesh=mesh,
        out_type=jax.ShapeDtypeStruct((BATCH_N, FDIM), jnp.float32),
        scratch_types=[
            pltpu.VMEM((_NCH, _CHUNK), jnp.int32),
            pltpu.VMEM((_NCH, _CHUNK), jnp.int32),
            pltpu.VMEM((_NCH, _CHUNK), jnp.int32),
            pltpu.VMEM((_BPW, FDIM), jnp.float32),
            pltpu.SemaphoreType.DMA,
        ],
        compiler_params=pltpu.CompilerParams(use_tc_tiling_on_sc=False),
    )(_gather_rows_body)


def _gather_rows_body(table_hbm, ids_hbm, count_hbm, out_hbm,
                      ids_v, cntg_v, idx_v, rows_v, sem):
    wid = lax.axis_index("s") * _NC + lax.axis_index("c")
    base = wid * _BPW
    for r in range(_NCH):
        pltpu.sync_copy(ids_hbm.at[pl.ds(base + r * _CHUNK, _CHUNK)],
                        ids_v.at[r])
    cnt_copies = [
        pltpu.async_copy(count_hbm.at[ids_v.at[r]], cntg_v.at[r], sem)
        for r in range(_NCH)
    ]
    for cp in cnt_copies:
        cp.wait()

    for r in range(_NCH):
        def body(j, _, r=r):
            sl = pl.ds(j * 16, 16)
            ids16 = ids_v[r, sl]
            cnt16 = cntg_v[r, sl]
            g = base + r * _CHUNK + j * 16 + lax.iota(jnp.int32, 16)
            idx_v[r, sl] = jnp.where(cnt16 > 0, ids16, N_CENTERS + g)
            return 0
        lax.fori_loop(0, _CHUNK // 16, body, 0)

    row_copies = [
        pltpu.async_copy(table_hbm.at[idx_v.at[r]],
                         rows_v.at[pl.ds(r * _CHUNK, _CHUNK)], sem)
        for r in range(_NCH)
    ]
    for cp in row_copies:
        cp.wait()

    pltpu.sync_copy(rows_v, out_hbm.at[pl.ds(base, _BPW)])


_FALLBACK = np.asarray(
    jax.random.normal(jax.random.key(1), (BATCH_N, FDIM), dtype=jnp.float32))


@jax.jit
def _impl(target_center_ids, bank, count):
    means = _compute_means(bank, count)
    table = jnp.concatenate([means, jnp.asarray(_FALLBACK)], axis=0)
    return _build_gather_rows()(table, target_center_ids, count)


def kernel(target_center_ids, bank, count):
    return _impl(target_center_ids, bank, count)


# speedup vs baseline: 10.8564x; 4.5684x over previous
import functools

import jax
import jax.numpy as jnp
import numpy as np
from jax import lax
from jax.experimental import pallas as pl
from jax.experimental.pallas import tpu as pltpu
from jax.experimental.pallas import tpu_sc as plsc

N_CENTERS = 10000
CAP = 100
FDIM = 64
BATCH_N = 16384

_NC = 2
_NS = 16
_NW = _NC * _NS

_TILE_L = 128
_SC_TILES = 32
_TC_NBLK = 78 - _SC_TILES
_SC_LANE0 = _TC_NBLK * _TILE_L
_SC_LANE1 = 78 * _TILE_L


C_BLK = _TILE_L


def _means_body(count_ref, bankt_ref, out_ref):
    cnt = count_ref[...]
    recip = 1.0 / jnp.maximum(cnt.astype(jnp.float32), 1.0)
    pos = lax.broadcasted_iota(jnp.int32, (CAP, 1, C_BLK), 0)
    w = jnp.where(pos < cnt[None], recip[None], 0.0)
    out_ref[...] = jnp.sum(bankt_ref[...] * w, axis=0)


def _tc_lane_block(i):
    return jnp.where(i < _TC_NBLK, i, 78)


def _compute_means_tc(bank_t, count):
    return pl.pallas_call(
        _means_body,
        grid=(_TC_NBLK + 1,),
        in_specs=[
            pl.BlockSpec((1, C_BLK), lambda i: (0, _tc_lane_block(i))),
            pl.BlockSpec((CAP, FDIM, C_BLK),
                         lambda i: (0, 0, _tc_lane_block(i))),
        ],
        out_specs=pl.BlockSpec((FDIM, C_BLK), lambda i: (0, _tc_lane_block(i))),
        out_shape=jax.ShapeDtypeStruct((FDIM, N_CENTERS), jnp.float32),
    )(count.reshape(1, N_CENTERS), bank_t)


_RB = 256
_NCHUNK = CAP * FDIM // _RB


def _sc_means_body(bank2_hbm, cnt8_hbm, out_hbm, cnt_v, buf_a, buf_b, acc_v,
                   sem_a, sem_b):
    wid = lax.axis_index("s") * _NC + lax.axis_index("c")

    @pl.when(wid < _SC_TILES)
    def _():
        lane0 = _SC_LANE0 + wid * _TILE_L
        pltpu.sync_copy(cnt8_hbm.at[pl.ds(0, 8), pl.ds(lane0, _TILE_L)],
                        cnt_v)
        cnt16 = [cnt_v[0, pl.ds(g * 16, 16)] for g in range(8)]

        def zero_row(f, _):
            z = jnp.zeros((16,), jnp.float32)
            for g in range(8):
                acc_v[f, pl.ds(g * 16, 16)] = z
            return 0
        lax.fori_loop(0, FDIM, zero_row, 0)

        bufs = (buf_a, buf_b)
        sems = (sem_a, sem_b)
        pend = [
            pltpu.async_copy(
                bank2_hbm.at[pl.ds(c * _RB, _RB), pl.ds(lane0, _TILE_L)],
                bufs[c], sems[c])
            for c in range(2)
        ]
        for c in range(_NCHUNK):
            k = c % 2
            buf, sem = bufs[k], sems[k]
            pend[k].wait()
            w = [[jnp.where(cnt16[g] > (4 * c + dj), 1.0, 0.0)
                  for g in range(8)] for dj in range(4)]

            def frow(f, _, w=w, buf=buf):
                for g in range(8):
                    sl = pl.ds(g * 16, 16)
                    a = acc_v[f, sl]
                    for dj in range(4):
                        a = a + buf[dj * FDIM + f, sl] * w[dj][g]
                    acc_v[f, sl] = a
                return 0
            lax.fori_loop(0, FDIM, frow, 0)
            if c + 2 < _NCHUNK:
                pend[k] = pltpu.async_copy(
                    bank2_hbm.at[pl.ds((c + 2) * _RB, _RB),
                                 pl.ds(lane0, _TILE_L)],
                    buf, sem)

        recip = [1.0 / jnp.maximum(cnt16[g].astype(jnp.float32), 1.0)
                 for g in range(8)]

        def scale_row(f, _):
            for g in range(8):
                sl = pl.ds(g * 16, 16)
                acc_v[f, sl] = acc_v[f, sl] * recip[g]
            return 0
        lax.fori_loop(0, FDIM, scale_row, 0)

        pltpu.sync_copy(acc_v,
                        out_hbm.at[pl.ds(0, FDIM), pl.ds(lane0, _TILE_L)])


@functools.cache
def _build_sc_means():
    mesh = plsc.VectorSubcoreMesh(core_axis_name="c", subcore_axis_name="s")
    return functools.partial(
        pl.kernel,
        mesh=mesh,
        out_type=jax.ShapeDtypeStruct((FDIM, N_CENTERS), jnp.float32),
        scratch_types=[
            pltpu.VMEM((8, _TILE_L), jnp.int32),
            pltpu.VMEM((_RB, _TILE_L), jnp.float32),
            pltpu.VMEM((_RB, _TILE_L), jnp.float32),
            pltpu.VMEM((FDIM, _TILE_L), jnp.float32),
            pltpu.SemaphoreType.DMA,
            pltpu.SemaphoreType.DMA,
        ],
    )(_sc_means_body)


_BPW = BATCH_N // _NW
_CHUNK = 128
_NCH = _BPW // _CHUNK


@functools.cache
def _build_gather_rows():
    mesh = plsc.VectorSubcoreMesh(core_axis_name="c", subcore_axis_name="s")
    return functools.partial(
        pl.kernel,
        mesh=mesh,
        out_type=jax.ShapeDtypeStruct((BATCH_N, FDIM), jnp.float32),
        scratch_types=[
            pltpu.VMEM((_NCH, _CHUNK), jnp.int32),
            pltpu.VMEM((_NCH, _CHUNK), jnp.int32),
            pltpu.VMEM((_NCH, _CHUNK), jnp.int32),
            pltpu.VMEM((_BPW, FDIM), jnp.float32),
            pltpu.SemaphoreType.DMA,
        ],
        compiler_params=pltpu.CompilerParams(use_tc_tiling_on_sc=False),
    )(_gather_rows_body)


def _gather_rows_body(table_hbm, ids_hbm, count_hbm, out_hbm,
                      ids_v, cntg_v, idx_v, rows_v, sem):
    wid = lax.axis_index("s") * _NC + lax.axis_index("c")
    base = wid * _BPW
    for r in range(_NCH):
        pltpu.sync_copy(ids_hbm.at[pl.ds(base + r * _CHUNK, _CHUNK)],
                        ids_v.at[r])
    cnt_copies = [
        pltpu.async_copy(count_hbm.at[ids_v.at[r]], cntg_v.at[r], sem)
        for r in range(_NCH)
    ]
    for cp in cnt_copies:
        cp.wait()

    for r in range(_NCH):
        def body(j, _, r=r):
            sl = pl.ds(j * 16, 16)
            ids16 = ids_v[r, sl]
            cnt16 = cntg_v[r, sl]
            g = base + r * _CHUNK + j * 16 + lax.iota(jnp.int32, 16)
            idx_v[r, sl] = jnp.where(cnt16 > 0, ids16, N_CENTERS + g)
            return 0
        lax.fori_loop(0, _CHUNK // 16, body, 0)

    row_copies = [
        pltpu.async_copy(table_hbm.at[idx_v.at[r]],
                         rows_v.at[pl.ds(r * _CHUNK, _CHUNK)], sem)
        for r in range(_NCH)
    ]
    for cp in row_copies:
        cp.wait()

    pltpu.sync_copy(rows_v, out_hbm.at[pl.ds(base, _BPW)])


_FALLBACK = np.asarray(
    jax.random.normal(jax.random.key(1), (BATCH_N, FDIM), dtype=jnp.float32))


@jax.jit
def _impl(target_center_ids, bank, count):
    bank_t = jnp.transpose(bank, (1, 2, 0))
    bank2 = bank_t.reshape(CAP * FDIM, N_CENTERS)
    cnt8 = jnp.broadcast_to(count[None, :], (8, N_CENTERS))
    sc_means = _build_sc_means()(bank2, cnt8)
    tc_means = _compute_means_tc(bank_t, count)
    means_t = jnp.concatenate(
        [tc_means[:, :_SC_LANE0], sc_means[:, _SC_LANE0:_SC_LANE1],
         tc_means[:, _SC_LANE1:]], axis=1)
    table = jnp.concatenate([means_t.T, jnp.asarray(_FALLBACK)], axis=0)
    return _build_gather_rows()(table, target_center_ids, count)


def kernel(target_center_ids, bank, count):
    return _impl(target_center_ids, bank, count)


# speedup vs baseline: 12.0899x; 1.1136x over previous
import functools

import jax
import jax.numpy as jnp
import numpy as np
from jax import lax
from jax.experimental import pallas as pl
from jax.experimental.pallas import tpu as pltpu
from jax.experimental.pallas import tpu_sc as plsc

N_CENTERS = 10000
CAP = 100
FDIM = 64
BATCH_N = 16384


C_BLK = 512


def _means_body(count_ref, bankt_ref, out_ref):
    cnt = count_ref[...]
    recip = 1.0 / jnp.maximum(cnt.astype(jnp.float32), 1.0)
    pos = lax.broadcasted_iota(jnp.int32, (CAP, 1, C_BLK), 0)
    w = jnp.where(pos < cnt[None], recip[None], 0.0)
    out_ref[...] = jnp.sum(bankt_ref[...] * w, axis=0)


def _compute_means_t(bank, count):
    bank_t = jnp.transpose(bank, (1, 2, 0))
    return pl.pallas_call(
        _means_body,
        grid=(pl.cdiv(N_CENTERS, C_BLK),),
        in_specs=[
            pl.BlockSpec((1, C_BLK), lambda i: (0, i)),
            pl.BlockSpec((CAP, FDIM, C_BLK), lambda i: (0, 0, i)),
        ],
        out_specs=pl.BlockSpec((FDIM, C_BLK), lambda i: (0, i)),
        out_shape=jax.ShapeDtypeStruct((FDIM, N_CENTERS), jnp.float32),
    )(count.reshape(1, N_CENTERS), bank_t)


_NC = 2
_NS = 16
_NW = _NC * _NS
_BPW = BATCH_N // _NW
_CHUNK = 128
_NCH = _BPW // _CHUNK

@functools.cache
def _build_gather_prep():
    mesh = plsc.VectorSubcoreMesh(core_axis_name="c", subcore_axis_name="s")
    return functools.partial(
        pl.kernel,
        mesh=mesh,
        out_type=jax.ShapeDtypeStruct((BATCH_N,), jnp.int32),
        scratch_types=[
            pltpu.VMEM((_NCH, _CHUNK), jnp.int32),
            pltpu.VMEM((_NCH, _CHUNK), jnp.int32),
            pltpu.VMEM((_NCH, _CHUNK), jnp.int32),
            pltpu.SemaphoreType.DMA,
        ],
        compiler_params=pltpu.CompilerParams(use_tc_tiling_on_sc=False),
    )(_gather_prep_body)


def _gather_prep_body(ids_hbm, count_hbm, idx_hbm, ids_v, cntg_v, idx_v, sem):
    wid = lax.axis_index("s") * _NC + lax.axis_index("c")
    base = wid * _BPW
    for r in range(_NCH):
        pltpu.sync_copy(ids_hbm.at[pl.ds(base + r * _CHUNK, _CHUNK)],
                        ids_v.at[r])
    cnt_copies = [
        pltpu.async_copy(count_hbm.at[ids_v.at[r]], cntg_v.at[r], sem)
        for r in range(_NCH)
    ]
    for cp in cnt_copies:
        cp.wait()

    for r in range(_NCH):
        def body(j, _, r=r):
            sl = pl.ds(j * 16, 16)
            ids16 = ids_v[r, sl]
            cnt16 = cntg_v[r, sl]
            g = base + r * _CHUNK + j * 16 + lax.iota(jnp.int32, 16)
            idx_v[r, sl] = jnp.where(cnt16 > 0, ids16, N_CENTERS + g)
            return 0
        lax.fori_loop(0, _CHUNK // 16, body, 0)

    for r in range(_NCH):
        pltpu.sync_copy(idx_v.at[r],
                        idx_hbm.at[pl.ds(base + r * _CHUNK, _CHUNK)])


@functools.cache
def _build_gather_rows():
    mesh = plsc.VectorSubcoreMesh(core_axis_name="c", subcore_axis_name="s")
    return functools.partial(
        pl.kernel,
        mesh=mesh,
        out_type=jax.ShapeDtypeStruct((BATCH_N, FDIM), jnp.float32),
        scratch_types=[
            pltpu.VMEM((_NCH, _CHUNK), jnp.int32),
            pltpu.VMEM((_BPW, FDIM), jnp.float32),
            pltpu.SemaphoreType.DMA,
        ],
        compiler_params=pltpu.CompilerParams(use_tc_tiling_on_sc=False),
    )(_gather_rows_body)


def _gather_rows_body(table_hbm, idx_hbm, out_hbm, idx_v, rows_v, sem):
    wid = lax.axis_index("s") * _NC + lax.axis_index("c")
    base = wid * _BPW
    for r in range(_NCH):
        pltpu.sync_copy(idx_hbm.at[pl.ds(base + r * _CHUNK, _CHUNK)],
                        idx_v.at[r])
    row_copies = [
        pltpu.async_copy(table_hbm.at[idx_v.at[r]],
                         rows_v.at[pl.ds(r * _CHUNK, _CHUNK)], sem)
        for r in range(_NCH)
    ]
    for cp in row_copies:
        cp.wait()

    pltpu.sync_copy(rows_v, out_hbm.at[pl.ds(base, _BPW)])


_FALLBACK = np.asarray(
    jax.random.normal(jax.random.key(1), (BATCH_N, FDIM), dtype=jnp.float32))


@jax.jit
def _impl(target_center_ids, bank, count):
    idx = _build_gather_prep()(target_center_ids, count)
    means_t = _compute_means_t(bank, count)
    table = jnp.concatenate([means_t.T, jnp.asarray(_FALLBACK)], axis=0)
    return _build_gather_rows()(table, idx)


def kernel(target_center_ids, bank, count):
    return _impl(target_center_ids, bank, count)


# speedup vs baseline: 12.1171x; 1.0023x over previous
import functools

import jax
import jax.numpy as jnp
import numpy as np
from jax import lax
from jax.experimental import pallas as pl
from jax.experimental.pallas import tpu as pltpu
from jax.experimental.pallas import tpu_sc as plsc

N_CENTERS = 10000
CAP = 100
FDIM = 64
BATCH_N = 16384


C_BLK = 512


def _means_body(count_ref, bankt_ref, out_ref):
    cnt = count_ref[...]
    recip = 1.0 / jnp.maximum(cnt.astype(jnp.float32), 1.0)
    pos = lax.broadcasted_iota(jnp.int32, (CAP, 1, C_BLK), 0)
    w = jnp.where(pos < cnt[None], recip[None], 0.0)
    out_ref[...] = jnp.sum(bankt_ref[...] * w, axis=0)


def _compute_means_t(bank, count):
    bank_t = jnp.transpose(bank, (1, 2, 0))
    return pl.pallas_call(
        _means_body,
        grid=(pl.cdiv(N_CENTERS, C_BLK),),
        in_specs=[
            pl.BlockSpec((1, C_BLK), lambda i: (0, i)),
            pl.BlockSpec((CAP, FDIM, C_BLK), lambda i: (0, 0, i)),
        ],
        out_specs=pl.BlockSpec((FDIM, C_BLK), lambda i: (0, i)),
        out_shape=jax.ShapeDtypeStruct((FDIM, N_CENTERS), jnp.float32),
    )(count.reshape(1, N_CENTERS), bank_t)


_NC = 2
_NS = 16
_NW = _NC * _NS
_BPW = BATCH_N // _NW
_CHUNK = 128
_NCH = _BPW // _CHUNK

@functools.cache
def _build_gather_rows():
    mesh = plsc.VectorSubcoreMesh(core_axis_name="c", subcore_axis_name="s")
    return functools.partial(
        pl.kernel,
        mesh=mesh,
        out_type=jax.ShapeDtypeStruct((BATCH_N, FDIM), jnp.float32),
        scratch_types=[
            pltpu.VMEM((_NCH, _CHUNK), jnp.int32),
            pltpu.VMEM((_NCH, _CHUNK), jnp.int32),
            pltpu.VMEM((_NCH, _CHUNK), jnp.int32),
            pltpu.VMEM((_BPW, FDIM), jnp.float32),
            pltpu.SemaphoreType.DMA,
        ],
        compiler_params=pltpu.CompilerParams(use_tc_tiling_on_sc=False),
    )(_gather_rows_body)


def _gather_rows_body(table_hbm, ids_hbm, count_hbm, out_hbm,
                      ids_v, cntg_v, idx_v, rows_v, sem):
    wid = lax.axis_index("s") * _NC + lax.axis_index("c")
    base = wid * _BPW
    for r in range(_NCH):
        pltpu.sync_copy(ids_hbm.at[pl.ds(base + r * _CHUNK, _CHUNK)],
                        ids_v.at[r])
    cnt_copies = [
        pltpu.async_copy(count_hbm.at[ids_v.at[r]], cntg_v.at[r], sem)
        for r in range(_NCH)
    ]
    for cp in cnt_copies:
        cp.wait()

    for r in range(_NCH):
        def body(j, _, r=r):
            sl = pl.ds(j * 16, 16)
            ids16 = ids_v[r, sl]
            cnt16 = cntg_v[r, sl]
            g = base + r * _CHUNK + j * 16 + lax.iota(jnp.int32, 16)
            idx_v[r, sl] = jnp.where(cnt16 > 0, ids16, N_CENTERS + g)
            return 0
        lax.fori_loop(0, _CHUNK // 16, body, 0)

    row_copies = [
        pltpu.async_copy(table_hbm.at[idx_v.at[r]],
                         rows_v.at[pl.ds(r * _CHUNK, _CHUNK)], sem)
        for r in range(_NCH)
    ]
    for cp in row_copies:
        cp.wait()

    pltpu.sync_copy(rows_v, out_hbm.at[pl.ds(base, _BPW)])


_FALLBACK = np.asarray(
    jax.random.normal(jax.random.key(1), (BATCH_N, FDIM), dtype=jnp.float32))


@jax.jit
def _impl(target_center_ids, bank, count):
    means_t = _compute_means_t(bank, count)
    table = jnp.concatenate([means_t.T, jnp.asarray(_FALLBACK)], axis=0)
    return _build_gather_rows()(table, target_center_ids, count)


def kernel(target_center_ids, bank, count):
    return _impl(target_center_ids, bank, count)
